# baseline (speedup 1.0000x reference)
import functools

import jax
import jax.numpy as jnp
from jax import lax
from jax.experimental import pallas as pl
from jax.experimental.pallas import tpu as pltpu
from jax.experimental.pallas import tpu_sc as plsc

N = 10000
NP = 10240
E = 320000
V = 4096
F = 128
H = 256
O = 128

NC = 2
NS = 16
NW = NC * NS
CH = 128
EPAD = 327680
ECHUNKS = EPAD // CH
EPW = ECHUNKS // NW
SEG = 40
RPS = NP // NS
XCH = 64
XPW = NP // XCH // NW

_mesh = plsc.VectorSubcoreMesh(
    core_axis_name="c", subcore_axis_name="s", num_cores=NC, num_subcores=NS
)


@functools.partial(
    pl.kernel,
    out_type=(
        jax.ShapeDtypeStruct((NC, NP, F), jnp.float32),
        jax.ShapeDtypeStruct((NP, H), jnp.float32),
    ),
    mesh=_mesh,
    scratch_types=[
        pltpu.VMEM((EPW, CH), jnp.int32),
        pltpu.VMEM((XPW, XCH), jnp.int32),
        pltpu.VMEM((CH, F), jnp.float32),
        pltpu.VMEM((XCH, H), jnp.float32),
        pltpu.VMEM_SHARED((NP, F), jnp.float32),
        pltpu.SemaphoreType.DMA,
        pltpu.SemaphoreType.DMA,
    ],
)
def _sc_deg_gather(dst3w, x3, zf, of, embw1, deg_out, h1pre, dstv, xv, onesv,
                   rowsv, acc, sem, semg):
    c = lax.axis_index("c")
    s = lax.axis_index("s")
    w = s * NC + c
    pltpu.sync_copy(zf.at[pl.ds(s * RPS, RPS)], acc.at[pl.ds(s * RPS, RPS)])
    pltpu.sync_copy(of, onesv)
    pltpu.sync_copy(dst3w.at[w], dstv)
    pltpu.sync_copy(x3.at[w], xv)
    plsc.subcore_barrier()
    for o in range(EPW // 8):
        for b in range(8):
            pltpu.async_copy(onesv, acc.at[dstv.at[o * 8 + b]], sem, add=True)
        if o < XPW:
            pltpu.async_copy(embw1.at[xv.at[o]], rowsv, semg).wait()
            pltpu.sync_copy(rowsv, h1pre.at[pl.ds((w * XPW + o) * XCH, XCH)])
        for b in range(8):
            pltpu.make_async_copy(of, onesv, sem).wait()
    plsc.subcore_barrier()
    pltpu.sync_copy(acc.at[pl.ds(s * RPS, RPS)],
                    deg_out.at[c, pl.ds(s * RPS, RPS)])


@functools.partial(
    pl.kernel,
    out_type=jax.ShapeDtypeStruct((NC, NP, F), jnp.float32),
    mesh=_mesh,
    scratch_types=[
        pltpu.VMEM((SEG, CH), jnp.int32),
        pltpu.VMEM((SEG, CH), jnp.int32),
        pltpu.VMEM((CH, F), jnp.float32),
        pltpu.VMEM((CH, F), jnp.float32),
        pltpu.VMEM_SHARED((NP, F), jnp.float32),
        pltpu.SemaphoreType.DMA,
        pltpu.SemaphoreType.DMA,
        pltpu.SemaphoreType.DMA,
        pltpu.SemaphoreType.DMA,
    ],
)
def _sc_agg1(src4s, dst4s, g1s, acc_out, srcv, dstv, rows0, rows1, acc,
             sem0, sem1, sems0, sems1):
    c = lax.axis_index("c")
    s = lax.axis_index("s")
    pltpu.sync_copy(g1s.at[c, pl.ds(s * RPS, RPS)], acc.at[pl.ds(s * RPS, RPS)])
    plsc.subcore_barrier()
    for seg in range(ECHUNKS // NS // SEG):
        pltpu.sync_copy(src4s.at[s, seg], srcv)
        pltpu.sync_copy(dst4s.at[s, seg], dstv)
        pltpu.async_copy(g1s.at[c].at[srcv.at[0]], rows0, sem0)

        @pl.loop(0, SEG, step=2)
        def _pair(j):
            pltpu.async_copy(g1s.at[c].at[srcv.at[j + 1]], rows1, sem1)
            pltpu.make_async_copy(g1s.at[c, pl.ds(0, CH)], rows0, sem0).wait()
            pltpu.async_copy(rows0, acc.at[dstv.at[j]], sems0, add=True)
            pltpu.make_async_copy(g1s.at[c, pl.ds(0, CH)], rows1, sem1).wait()
            pltpu.async_copy(rows1, acc.at[dstv.at[j + 1]], sems1, add=True)
            jn = jnp.minimum(j + 2, SEG - 1)
            pltpu.make_async_copy(g1s.at[c, pl.ds(0, CH)], rows0, sems0).wait()
            pltpu.async_copy(g1s.at[c].at[srcv.at[jn]], rows0, sem0)
            pltpu.make_async_copy(g1s.at[c, pl.ds(0, CH)], rows1, sems1).wait()

        pltpu.make_async_copy(g1s.at[c, pl.ds(0, CH)], rows0, sem0).wait()
    plsc.subcore_barrier()
    pltpu.sync_copy(acc.at[pl.ds(s * RPS, RPS)],
                    acc_out.at[c, pl.ds(s * RPS, RPS)])


@functools.partial(
    pl.kernel,
    out_type=jax.ShapeDtypeStruct((NC, NP, O), jnp.float32),
    mesh=_mesh,
    scratch_types=[
        pltpu.VMEM((SEG, CH), jnp.int32),
        pltpu.VMEM((SEG, CH), jnp.int32),
        pltpu.VMEM((CH, O), jnp.float32),
        pltpu.VMEM((CH, O), jnp.float32),
        pltpu.VMEM_SHARED((NP, O), jnp.float32),
        pltpu.SemaphoreType.DMA,
        pltpu.SemaphoreType.DMA,
        pltpu.SemaphoreType.DMA,
        pltpu.SemaphoreType.DMA,
    ],
)
def _sc_agg2(src4w, dst4w, init2, acc_out, srcv, dstv, rows0, rows1, acc,
             sem0, sem1, sems0, sems1):
    c = lax.axis_index("c")
    s = lax.axis_index("s")
    w = s * NC + c
    pltpu.sync_copy(init2.at[c, pl.ds(s * RPS, RPS)], acc.at[pl.ds(s * RPS, RPS)])
    plsc.subcore_barrier()
    for seg in range(EPW // SEG):
        pltpu.sync_copy(src4w.at[w, seg], srcv)
        pltpu.sync_copy(dst4w.at[w, seg], dstv)
        pltpu.async_copy(init2.at[0].at[srcv.at[0]], rows0, sem0)

        @pl.loop(0, SEG, step=2)
        def _pair(j):
            pltpu.async_copy(init2.at[0].at[srcv.at[j + 1]], rows1, sem1)
            pltpu.make_async_copy(init2.at[0, pl.ds(0, CH)], rows0, sem0).wait()
            pltpu.async_copy(rows0, acc.at[dstv.at[j]], sems0, add=True)
            pltpu.make_async_copy(init2.at[0, pl.ds(0, CH)], rows1, sem1).wait()
            pltpu.async_copy(rows1, acc.at[dstv.at[j + 1]], sems1, add=True)
            jn = jnp.minimum(j + 2, SEG - 1)
            pltpu.make_async_copy(init2.at[0, pl.ds(0, CH)], rows0, sems0).wait()
            pltpu.async_copy(init2.at[0].at[srcv.at[jn]], rows0, sem0)
            pltpu.make_async_copy(init2.at[0, pl.ds(0, CH)], rows1, sems1).wait()

        pltpu.make_async_copy(init2.at[0, pl.ds(0, CH)], rows0, sem0).wait()
    plsc.subcore_barrier()
    pltpu.sync_copy(acc.at[pl.ds(s * RPS, RPS)],
                    acc_out.at[c, pl.ds(s * RPS, RPS)])


def _tc_embw1_body(emb_ref, w1_ref, out_ref):
    out_ref[...] = jnp.dot(emb_ref[...], w1_ref[...],
                           preferred_element_type=jnp.float32)


def _tc_embw1(emb, w1):
    return pl.pallas_call(
        _tc_embw1_body,
        out_shape=jax.ShapeDtypeStruct((V, H), jnp.float32),
    )(emb, w1)


def _tc_norm1_body(deg_ref, h1pre_ref, dis_ref, g1_ref):
    d = deg_ref[0][:, :16] + deg_ref[1][:, :16] + 1.0
    dis = lax.rsqrt(d)
    dis_ref[...] = dis
    g = h1pre_ref[...] * dis[:, :1]
    g1_ref[...] = jnp.stack([g[:, :F], g[:, F:]])


def _tc_norm1(deg16, h1pre):
    blk = 512
    return pl.pallas_call(
        _tc_norm1_body,
        grid=(NP // blk,),
        in_specs=[
            pl.BlockSpec((NC, blk, F), lambda i: (0, i, 0)),
            pl.BlockSpec((blk, H), lambda i: (i, 0)),
        ],
        out_specs=[
            pl.BlockSpec((blk, 16), lambda i: (i, 0)),
            pl.BlockSpec((2, blk, F), lambda i: (0, i, 0)),
        ],
        out_shape=[
            jax.ShapeDtypeStruct((NP, 16), jnp.float32),
            jax.ShapeDtypeStruct((2, NP, F), jnp.float32),
        ],
    )(deg16, h1pre)


def _tc_layer2_body(acc1_ref, dis_ref, b1_ref, w2_ref, init2_ref):
    dis = dis_ref[...][:, :1]
    h1 = jnp.concatenate([acc1_ref[0], acc1_ref[1]], axis=1)
    h1 = jnp.maximum(dis * h1 + b1_ref[...], 0.0)
    g2 = jnp.dot(h1, w2_ref[...], preferred_element_type=jnp.float32) * dis
    init2_ref[...] = jnp.stack([g2, jnp.zeros_like(g2)])


def _tc_layer2(acc1, dis16, b1r, w2):
    blk = 512
    return pl.pallas_call(
        _tc_layer2_body,
        grid=(NP // blk,),
        in_specs=[
            pl.BlockSpec((NC, blk, F), lambda i: (0, i, 0)),
            pl.BlockSpec((blk, 16), lambda i: (i, 0)),
            pl.BlockSpec((1, H), lambda i: (0, 0)),
            pl.BlockSpec((H, O), lambda i: (0, 0)),
        ],
        out_specs=pl.BlockSpec((2, blk, O), lambda i: (0, i, 0)),
        out_shape=jax.ShapeDtypeStruct((2, NP, O), jnp.float32),
    )(acc1, dis16, b1r, w2)


def _tc_head_body(acc2_ref, dis_ref, b2_ref, wfc_ref, bfc_ref, out_ref):
    dis = dis_ref[...][:, :1]
    h2 = jnp.maximum(dis * (acc2_ref[0] + acc2_ref[1]) + b2_ref[...], 0.0)
    out_ref[...] = jnp.dot(h2, wfc_ref[...],
                           preferred_element_type=jnp.float32) + bfc_ref[...]


def _tc_head(acc2, dis16, b2r, wfc, bfcr):
    nblk, vblk = 1000, 512
    return pl.pallas_call(
        _tc_head_body,
        grid=(N // nblk, V // vblk),
        in_specs=[
            pl.BlockSpec((NC, nblk, O), lambda i, j: (0, i, 0)),
            pl.BlockSpec((nblk, 16), lambda i, j: (i, 0)),
            pl.BlockSpec((1, O), lambda i, j: (0, 0)),
            pl.BlockSpec((O, vblk), lambda i, j: (0, j)),
            pl.BlockSpec((1, vblk), lambda i, j: (0, j)),
        ],
        out_specs=pl.BlockSpec((nblk, vblk), lambda i, j: (i, j)),
        out_shape=jax.ShapeDtypeStruct((N, V), jnp.float32),
    )(acc2, dis16, b2r, wfc, bfcr)


@jax.jit
def kernel(x, edge_index, emb, W1, b1, W2, b2, Wfc, bfc):
    xpad = jnp.concatenate(
        [x.astype(jnp.int32), jnp.arange(NP - N, dtype=jnp.int32) % V])
    x3 = xpad.reshape(NW, XPW, XCH)
    src = edge_index[0].astype(jnp.int32)
    dst = edge_index[1].astype(jnp.int32)
    npad = EPAD - E
    pad_src = jnp.arange(npad, dtype=jnp.int32) % N
    pad_dst = N + (jnp.arange(npad, dtype=jnp.int32) % (NP - N))
    srcp = jnp.concatenate([src, pad_src])
    dstp = jnp.concatenate([dst, pad_dst])
    dst3w = dstp.reshape(NW, EPW, CH)
    src4w = srcp.reshape(NW, EPW // SEG, SEG, CH)
    dst4w = dstp.reshape(NW, EPW // SEG, SEG, CH)
    src4s = srcp.reshape(NS, ECHUNKS // NS // SEG, SEG, CH)
    dst4s = dstp.reshape(NS, ECHUNKS // NS // SEG, SEG, CH)
    zf = jnp.zeros((NP, F), jnp.float32)
    of = jnp.ones((CH, F), jnp.float32)
    b1r = b1.reshape(1, H)
    b2r = b2.reshape(1, O)
    bfcr = bfc.reshape(1, V)

    embw1 = _tc_embw1(emb, W1)
    deg16, h1pre = _sc_deg_gather(dst3w, x3, zf, of, embw1)
    dis16, g1s = _tc_norm1(deg16, h1pre)
    acc1 = _sc_agg1(src4s, dst4s, g1s)
    init2 = _tc_layer2(acc1, dis16, b1r, W2)
    acc2 = _sc_agg2(src4w, dst4w, init2)
    return _tc_head(acc2, dis16, b2r, Wfc, bfcr)


# speedup vs baseline: 1.1978x; 1.1978x over previous
import functools

import jax
import jax.numpy as jnp
from jax import lax
from jax.experimental import pallas as pl
from jax.experimental.pallas import tpu as pltpu
from jax.experimental.pallas import tpu_sc as plsc

N = 10000
NP = 10240
E = 320000
V = 4096
F = 128
H = 256
O = 128

NC = 2
NS = 16
NW = NC * NS
CH = 128
EPAD = 327680
ECHUNKS = EPAD // CH
EPW = ECHUNKS // NW
SEG = 40
RPS = NP // NS
XCH = 64
XPW = NP // XCH // NW

_mesh = plsc.VectorSubcoreMesh(
    core_axis_name="c", subcore_axis_name="s", num_cores=NC, num_subcores=NS
)


@functools.partial(
    pl.kernel,
    out_type=(
        jax.ShapeDtypeStruct((NC, NP, F), jnp.float32),
        jax.ShapeDtypeStruct((NP, H), jnp.float32),
    ),
    mesh=_mesh,
    scratch_types=[
        pltpu.VMEM((EPW, CH), jnp.int32),
        pltpu.VMEM((XPW, XCH), jnp.int32),
        pltpu.VMEM((CH, F), jnp.float32),
        pltpu.VMEM((XCH, H), jnp.float32),
        pltpu.VMEM_SHARED((NP, F), jnp.float32),
        pltpu.SemaphoreType.DMA,
        pltpu.SemaphoreType.DMA,
    ],
)
def _sc_deg_gather(dst3w, x3, zf, of, embw1, deg_out, h1pre, dstv, xv, onesv,
                   rowsv, acc, sem, semg):
    c = lax.axis_index("c")
    s = lax.axis_index("s")
    w = s * NC + c
    pltpu.sync_copy(zf.at[pl.ds(s * RPS, RPS)], acc.at[pl.ds(s * RPS, RPS)])
    pltpu.sync_copy(of, onesv)
    pltpu.sync_copy(dst3w.at[w], dstv)
    pltpu.sync_copy(x3.at[w], xv)
    plsc.subcore_barrier()
    for o in range(EPW // 8):
        for b in range(8):
            pltpu.async_copy(onesv, acc.at[dstv.at[o * 8 + b]], sem, add=True)
        if o < XPW:
            pltpu.async_copy(embw1.at[xv.at[o]], rowsv, semg).wait()
            pltpu.sync_copy(rowsv, h1pre.at[pl.ds((w * XPW + o) * XCH, XCH)])
        for b in range(8):
            pltpu.make_async_copy(of, onesv, sem).wait()
    plsc.subcore_barrier()
    pltpu.sync_copy(acc.at[pl.ds(s * RPS, RPS)],
                    deg_out.at[c, pl.ds(s * RPS, RPS)])


@functools.partial(
    pl.kernel,
    out_type=jax.ShapeDtypeStruct((NC, NP, F), jnp.float32),
    mesh=_mesh,
    scratch_types=[
        pltpu.VMEM((SEG, CH), jnp.int32),
        pltpu.VMEM((SEG, CH), jnp.int32),
        pltpu.VMEM((CH, F), jnp.float32),
        pltpu.VMEM((CH, F), jnp.float32),
        pltpu.VMEM_SHARED((NP, F), jnp.float32),
        pltpu.SemaphoreType.DMA,
        pltpu.SemaphoreType.DMA,
    ],
)
def _sc_agg1(src4s, dst4s, g1s, acc_out, srcv, dstv, rows0, rows1, acc,
             sem0, sem1):
    c = lax.axis_index("c")
    s = lax.axis_index("s")
    pltpu.sync_copy(g1s.at[c, pl.ds(s * RPS, RPS)], acc.at[pl.ds(s * RPS, RPS)])
    plsc.subcore_barrier()
    for seg in range(ECHUNKS // NS // SEG):
        pltpu.sync_copy(src4s.at[s, seg], srcv)
        pltpu.sync_copy(dst4s.at[s, seg], dstv)
        pltpu.async_copy(g1s.at[c].at[srcv.at[0]], rows0, sem0)

        @pl.loop(0, SEG, step=2)
        def _pair(j):
            pltpu.async_copy(g1s.at[c].at[srcv.at[j + 1]], rows1, sem1)
            pltpu.make_async_copy(g1s.at[c, pl.ds(0, CH)], rows0, sem0).wait()
            pltpu.sync_copy(rows0, acc.at[dstv.at[j]], add=True)
            jn = jnp.minimum(j + 2, SEG - 1)
            pltpu.async_copy(g1s.at[c].at[srcv.at[jn]], rows0, sem0)
            pltpu.make_async_copy(g1s.at[c, pl.ds(0, CH)], rows1, sem1).wait()
            pltpu.sync_copy(rows1, acc.at[dstv.at[j + 1]], add=True)

        pltpu.make_async_copy(g1s.at[c, pl.ds(0, CH)], rows0, sem0).wait()
    plsc.subcore_barrier()
    pltpu.sync_copy(acc.at[pl.ds(s * RPS, RPS)],
                    acc_out.at[c, pl.ds(s * RPS, RPS)])


@functools.partial(
    pl.kernel,
    out_type=jax.ShapeDtypeStruct((NC, NP, O), jnp.float32),
    mesh=_mesh,
    scratch_types=[
        pltpu.VMEM((SEG, CH), jnp.int32),
        pltpu.VMEM((SEG, CH), jnp.int32),
        pltpu.VMEM((CH, O), jnp.float32),
        pltpu.VMEM((CH, O), jnp.float32),
        pltpu.VMEM_SHARED((NP, O), jnp.float32),
        pltpu.SemaphoreType.DMA,
        pltpu.SemaphoreType.DMA,
    ],
)
def _sc_agg2(src4w, dst4w, init2, acc_out, srcv, dstv, rows0, rows1, acc,
             sem0, sem1):
    c = lax.axis_index("c")
    s = lax.axis_index("s")
    w = s * NC + c
    pltpu.sync_copy(init2.at[c, pl.ds(s * RPS, RPS)], acc.at[pl.ds(s * RPS, RPS)])
    plsc.subcore_barrier()
    for seg in range(EPW // SEG):
        pltpu.sync_copy(src4w.at[w, seg], srcv)
        pltpu.sync_copy(dst4w.at[w, seg], dstv)
        pltpu.async_copy(init2.at[0].at[srcv.at[0]], rows0, sem0)

        @pl.loop(0, SEG, step=2)
        def _pair(j):
            pltpu.async_copy(init2.at[0].at[srcv.at[j + 1]], rows1, sem1)
            pltpu.make_async_copy(init2.at[0, pl.ds(0, CH)], rows0, sem0).wait()
            pltpu.sync_copy(rows0, acc.at[dstv.at[j]], add=True)
            jn = jnp.minimum(j + 2, SEG - 1)
            pltpu.async_copy(init2.at[0].at[srcv.at[jn]], rows0, sem0)
            pltpu.make_async_copy(init2.at[0, pl.ds(0, CH)], rows1, sem1).wait()
            pltpu.sync_copy(rows1, acc.at[dstv.at[j + 1]], add=True)

        pltpu.make_async_copy(init2.at[0, pl.ds(0, CH)], rows0, sem0).wait()
    plsc.subcore_barrier()
    pltpu.sync_copy(acc.at[pl.ds(s * RPS, RPS)],
                    acc_out.at[c, pl.ds(s * RPS, RPS)])


def _tc_embw1_body(emb_ref, w1_ref, out_ref):
    out_ref[...] = jnp.dot(emb_ref[...], w1_ref[...],
                           preferred_element_type=jnp.float32)


def _tc_embw1(emb, w1):
    return pl.pallas_call(
        _tc_embw1_body,
        out_shape=jax.ShapeDtypeStruct((V, H), jnp.float32),
    )(emb, w1)


def _tc_norm1_body(deg_ref, h1pre_ref, dis_ref, g1_ref):
    d = deg_ref[0][:, :16] + deg_ref[1][:, :16] + 1.0
    dis = lax.rsqrt(d)
    dis_ref[...] = dis
    g = h1pre_ref[...] * dis[:, :1]
    g1_ref[...] = jnp.stack([g[:, :F], g[:, F:]])


def _tc_norm1(deg16, h1pre):
    blk = 512
    return pl.pallas_call(
        _tc_norm1_body,
        grid=(NP // blk,),
        in_specs=[
            pl.BlockSpec((NC, blk, F), lambda i: (0, i, 0)),
            pl.BlockSpec((blk, H), lambda i: (i, 0)),
        ],
        out_specs=[
            pl.BlockSpec((blk, 16), lambda i: (i, 0)),
            pl.BlockSpec((2, blk, F), lambda i: (0, i, 0)),
        ],
        out_shape=[
            jax.ShapeDtypeStruct((NP, 16), jnp.float32),
            jax.ShapeDtypeStruct((2, NP, F), jnp.float32),
        ],
    )(deg16, h1pre)


def _tc_layer2_body(acc1_ref, dis_ref, b1_ref, w2_ref, init2_ref):
    dis = dis_ref[...][:, :1]
    h1 = jnp.concatenate([acc1_ref[0], acc1_ref[1]], axis=1)
    h1 = jnp.maximum(dis * h1 + b1_ref[...], 0.0)
    g2 = jnp.dot(h1, w2_ref[...], preferred_element_type=jnp.float32) * dis
    init2_ref[...] = jnp.stack([g2, jnp.zeros_like(g2)])


def _tc_layer2(acc1, dis16, b1r, w2):
    blk = 512
    return pl.pallas_call(
        _tc_layer2_body,
        grid=(NP // blk,),
        in_specs=[
            pl.BlockSpec((NC, blk, F), lambda i: (0, i, 0)),
            pl.BlockSpec((blk, 16), lambda i: (i, 0)),
            pl.BlockSpec((1, H), lambda i: (0, 0)),
            pl.BlockSpec((H, O), lambda i: (0, 0)),
        ],
        out_specs=pl.BlockSpec((2, blk, O), lambda i: (0, i, 0)),
        out_shape=jax.ShapeDtypeStruct((2, NP, O), jnp.float32),
    )(acc1, dis16, b1r, w2)


def _tc_head_body(acc2_ref, dis_ref, b2_ref, wfc_ref, bfc_ref, out_ref):
    dis = dis_ref[...][:, :1]
    h2 = jnp.maximum(dis * (acc2_ref[0] + acc2_ref[1]) + b2_ref[...], 0.0)
    out_ref[...] = jnp.dot(h2, wfc_ref[...],
                           preferred_element_type=jnp.float32) + bfc_ref[...]


def _tc_head(acc2, dis16, b2r, wfc, bfcr):
    nblk, vblk = 1000, 512
    return pl.pallas_call(
        _tc_head_body,
        grid=(N // nblk, V // vblk),
        in_specs=[
            pl.BlockSpec((NC, nblk, O), lambda i, j: (0, i, 0)),
            pl.BlockSpec((nblk, 16), lambda i, j: (i, 0)),
            pl.BlockSpec((1, O), lambda i, j: (0, 0)),
            pl.BlockSpec((O, vblk), lambda i, j: (0, j)),
            pl.BlockSpec((1, vblk), lambda i, j: (0, j)),
        ],
        out_specs=pl.BlockSpec((nblk, vblk), lambda i, j: (i, j)),
        out_shape=jax.ShapeDtypeStruct((N, V), jnp.float32),
    )(acc2, dis16, b2r, wfc, bfcr)


@jax.jit
def kernel(x, edge_index, emb, W1, b1, W2, b2, Wfc, bfc):
    xpad = jnp.concatenate(
        [x.astype(jnp.int32), jnp.arange(NP - N, dtype=jnp.int32) % V])
    x3 = xpad.reshape(NW, XPW, XCH)
    src = edge_index[0].astype(jnp.int32)
    dst = edge_index[1].astype(jnp.int32)
    npad = EPAD - E
    pad_src = jnp.arange(npad, dtype=jnp.int32) % N
    pad_dst = N + (jnp.arange(npad, dtype=jnp.int32) % (NP - N))
    srcp = jnp.concatenate([src, pad_src])
    dstp = jnp.concatenate([dst, pad_dst])
    dst3w = dstp.reshape(NW, EPW, CH)
    src4w = srcp.reshape(NW, EPW // SEG, SEG, CH)
    dst4w = dstp.reshape(NW, EPW // SEG, SEG, CH)
    src4s = srcp.reshape(NS, ECHUNKS // NS // SEG, SEG, CH)
    dst4s = dstp.reshape(NS, ECHUNKS // NS // SEG, SEG, CH)
    zf = jnp.zeros((NP, F), jnp.float32)
    of = jnp.ones((CH, F), jnp.float32)
    b1r = b1.reshape(1, H)
    b2r = b2.reshape(1, O)
    bfcr = bfc.reshape(1, V)

    embw1 = _tc_embw1(emb, W1)
    deg16, h1pre = _sc_deg_gather(dst3w, x3, zf, of, embw1)
    dis16, g1s = _tc_norm1(deg16, h1pre)
    acc1 = _sc_agg1(src4s, dst4s, g1s)
    init2 = _tc_layer2(acc1, dis16, b1r, W2)
    acc2 = _sc_agg2(src4w, dst4w, init2)
    return _tc_head(acc2, dis16, b2r, Wfc, bfcr)
